# baseline (speedup 1.0000x reference)
import jax
import jax.numpy as jnp
from jax.experimental import pallas as pl


def _flatten_block(x_ref, o_ref):
    blk = x_ref.shape[0]
    o_ref[...] = x_ref[...].reshape(blk, -1)


def kernel(embedded_tasks):
    b, c, h, w = embedded_tasks.shape
    f = c * h * w
    blk = 64
    return pl.pallas_call(
        _flatten_block,
        grid=(b // blk,),
        in_specs=[pl.BlockSpec((blk, c, h, w), lambda i: (i, 0, 0, 0))],
        out_specs=pl.BlockSpec((blk, f), lambda i: (i, 0)),
        out_shape=jax.ShapeDtypeStruct((b, f), jnp.float32),
    )(embedded_tasks)
